# baseline (speedup 1.0000x reference)
import math

import jax
import jax.numpy as jnp
import ml_dtypes
import numpy as np
from jax import lax
from jax.experimental import pallas as pl
from jax.experimental.pallas import tpu as pltpu

_CONST_CACHE = {}


def _tf2x32(k1, k2, x0, x1):
    k1 = np.uint32(k1)
    k2 = np.uint32(k2)
    x0 = x0.astype(np.uint32).copy()
    x1 = x1.astype(np.uint32).copy()
    kx = np.uint32(k1 ^ k2 ^ np.uint32(0x1BD11BDA))
    rot1 = (13, 15, 26, 6)
    rot2 = (17, 29, 16, 24)
    ks = (k1, k2, kx)

    def rol(v, d):
        return (v << np.uint32(d)) | (v >> np.uint32(32 - d))

    with np.errstate(over="ignore"):
        x0 += ks[0]
        x1 += ks[1]
        rots = (rot1, rot2, rot1, rot2, rot1)
        for i in range(5):
            for r in rots[i]:
                x0 += x1
                x1 = rol(x1, r)
                x1 ^= x0
            x0 += ks[(i + 1) % 3]
            x1 += ks[(i + 2) % 3] + np.uint32(i + 1)
    return x0, x1


def _sample_indices(L, S, u):
    a, b = _tf2x32(np.uint32(0), np.uint32(42),
                   np.array([0], np.uint32), np.array([7], np.uint32))
    k1, k2 = a[0], b[0]
    c1, c2 = _tf2x32(k1, k2, np.zeros(2, np.uint32),
                     np.arange(2, dtype=np.uint32))
    lo_key = (c1[1], c2[1])
    n = L * u
    hi = np.zeros(n, np.uint32)
    lo = np.arange(n, dtype=np.uint32)
    b1, b2 = _tf2x32(lo_key[0], lo_key[1], hi, lo)
    bits = b1 ^ b2
    return (bits % np.uint32(S)).astype(np.int32).reshape(L, u)


def _mask_matrices(L, S, u):
    ck = (L, S, u)
    if ck not in _CONST_CACHE:
        idx = _sample_indices(L, S, u)
        C = np.zeros((L, S), np.float32)
        np.add.at(C, (np.arange(L)[:, None], idx), 1.0)
        A = np.where(C > 0, np.float32(0), np.float32(-np.inf))
        _CONST_CACHE[ck] = (C.T.copy().astype(ml_dtypes.bfloat16),
                            A.T.copy().astype(ml_dtypes.bfloat16))
    return _CONST_CACHE[ck]


def _make_m_topk_body(BH, U):
    def _body(q_ref, k_ref, cf_ref, madd_ref, mt_ref, m_sc):
        L, D = q_ref.shape[1], q_ref.shape[2]
        S = k_ref.shape[1]
        LB = 512
        neg = jnp.float32(-jnp.inf)
        pid = pl.program_id(0)

        @pl.when(pid < BH)
        def _phase_m():
            kk = k_ref[0]
            cdims = (((1,), (1,)), ((), ()))
            for r in range(L // LB):
                sl = slice(LB * r, LB * (r + 1))
                qkT = lax.dot_general(kk, q_ref[0, sl, :], cdims,
                                      preferred_element_type=jnp.float32)
                madd = madd_ref[:, sl].astype(jnp.float32)
                cf = cf_ref[:, sl].astype(jnp.float32)
                mmax = jnp.max(qkT + madd, axis=0, keepdims=True)
                msum = jnp.sum(qkT * cf, axis=0, keepdims=True)
                m_sc[pl.ds(pid, 1), sl] = mmax - msum / jnp.float32(S)

        @pl.when(pid == BH)
        def _phase_topk():
            mv = m_sc[:, :]
            lin = lax.broadcasted_iota(jnp.int32, (BH, L), 1)
            for i in range(U):
                rmax = jnp.max(mv, axis=1, keepdims=True)
                idxc = jnp.min(jnp.where(mv == rmax, lin, S),
                               axis=1, keepdims=True)
                mt_ref[:, pl.ds(i, 1), :] = jnp.broadcast_to(
                    idxc.astype(jnp.float32)[:, :, None], (BH, 1, 128))
                mv = jnp.where(lin == idxc, neg, mv)

    return _body


def _attn_body(q_ref, k_ref, v_ref, mt_ref, o_ref):
    D = q_ref.shape[2]
    S = k_ref.shape[1]
    U = mt_ref.shape[1]
    neg = jnp.float32(-jnp.inf)
    scale = jnp.float32(1.0 / math.sqrt(D))
    kk = k_ref[0]
    vv = v_ref[0]

    mtopf = mt_ref[0, :, 0:1]
    colf = lax.broadcasted_iota(jnp.int32, (U, S), 1).astype(jnp.float32)
    oh = (colf == mtopf).astype(jnp.float32)

    qsel = lax.dot_general(oh, q_ref[0], (((1,), (0,)), ((), ())),
                           preferred_element_type=jnp.float32)
    sc = lax.dot_general(qsel, kk, (((1,), (1,)), ((), ())),
                         preferred_element_type=jnp.float32) * scale
    scm = jnp.where(colf <= mtopf, sc, neg)
    rmax = jnp.max(scm, axis=1, keepdims=True)
    p = jnp.exp(scm - rmax)
    attn = p / jnp.sum(p, axis=1, keepdims=True)
    upd = lax.dot_general(attn, vv, (((1,), (0,)), ((), ())),
                          preferred_element_type=jnp.float32)

    tri = (lax.broadcasted_iota(jnp.int32, (D, D), 0)
           <= lax.broadcasted_iota(jnp.int32, (D, D), 1))
    T = jnp.where(tri, jnp.float32(1.0), jnp.float32(0.0))
    cumvT = lax.dot_general(T, vv, (((0,), (1,)), ((), ())),
                            preferred_element_type=jnp.float32)

    merged = jnp.concatenate([upd, jnp.ones((U, 1), jnp.float32)], axis=1)
    fullT = lax.dot_general(merged, oh, (((0,), (0,)), ((), ())),
                            preferred_element_type=jnp.float32)
    o_ref[0, 0, :, :] = jnp.where(fullT[D:D + 1, :] > 0.5, fullT[:D, :],
                                  cumvT)


def kernel(queries, keys, values, attn_mask):
    B, L, H, D = queries.shape
    S = keys.shape[1]
    BH = B * H
    factor = 5
    U = factor * int(np.ceil(np.log(S)))
    u = factor * int(np.ceil(np.log(L)))
    Cnp, Anp = _mask_matrices(L, S, u)
    CF = jnp.asarray(Cnp)
    MADD = jnp.asarray(Anp)

    q = queries.reshape(BH, L, D)
    k = keys.reshape(BH, S, D)
    v = values.reshape(BH, S, D)

    mt = pl.pallas_call(
        _make_m_topk_body(BH, U),
        grid=(BH + 1,),
        in_specs=[
            pl.BlockSpec((1, L, D), lambda j: (jnp.minimum(j, BH - 1), 0, 0)),
            pl.BlockSpec((1, S, D), lambda j: (jnp.minimum(j, BH - 1), 0, 0)),
            pl.BlockSpec((S, L), lambda j: (0, 0)),
            pl.BlockSpec((S, L), lambda j: (0, 0)),
        ],
        out_specs=pl.BlockSpec((BH, U, 128), lambda j: (0, 0, 0)),
        out_shape=jax.ShapeDtypeStruct((BH, U, 128), jnp.float32),
        scratch_shapes=[pltpu.VMEM((BH, L), jnp.float32)],
    )(q, k, CF, MADD)

    out = pl.pallas_call(
        _attn_body,
        grid=(BH,),
        in_specs=[
            pl.BlockSpec((1, L, D), lambda j: (j, 0, 0)),
            pl.BlockSpec((1, S, D), lambda j: (j, 0, 0)),
            pl.BlockSpec((1, S, D), lambda j: (j, 0, 0)),
            pl.BlockSpec((1, U, 128), lambda j: (j, 0, 0)),
        ],
        out_specs=pl.BlockSpec(
            (1, 1, D, S), lambda j: (j // H, j % H, 0, 0)),
        out_shape=jax.ShapeDtypeStruct((B, H, D, S), jnp.float32),
    )(q, k, v, mt)
    return jnp.swapaxes(out, 2, 3)


# speedup vs baseline: 1.0016x; 1.0016x over previous
import math

import jax
import jax.numpy as jnp
import ml_dtypes
import numpy as np
from jax import lax
from jax.experimental import pallas as pl
from jax.experimental.pallas import tpu as pltpu

_CONST_CACHE = {}


def _tf2x32(k1, k2, x0, x1):
    k1 = np.uint32(k1)
    k2 = np.uint32(k2)
    x0 = x0.astype(np.uint32).copy()
    x1 = x1.astype(np.uint32).copy()
    kx = np.uint32(k1 ^ k2 ^ np.uint32(0x1BD11BDA))
    rot1 = (13, 15, 26, 6)
    rot2 = (17, 29, 16, 24)
    ks = (k1, k2, kx)

    def rol(v, d):
        return (v << np.uint32(d)) | (v >> np.uint32(32 - d))

    with np.errstate(over="ignore"):
        x0 += ks[0]
        x1 += ks[1]
        rots = (rot1, rot2, rot1, rot2, rot1)
        for i in range(5):
            for r in rots[i]:
                x0 += x1
                x1 = rol(x1, r)
                x1 ^= x0
            x0 += ks[(i + 1) % 3]
            x1 += ks[(i + 2) % 3] + np.uint32(i + 1)
    return x0, x1


def _sample_indices(L, S, u):
    a, b = _tf2x32(np.uint32(0), np.uint32(42),
                   np.array([0], np.uint32), np.array([7], np.uint32))
    k1, k2 = a[0], b[0]
    c1, c2 = _tf2x32(k1, k2, np.zeros(2, np.uint32),
                     np.arange(2, dtype=np.uint32))
    lo_key = (c1[1], c2[1])
    n = L * u
    hi = np.zeros(n, np.uint32)
    lo = np.arange(n, dtype=np.uint32)
    b1, b2 = _tf2x32(lo_key[0], lo_key[1], hi, lo)
    bits = b1 ^ b2
    return (bits % np.uint32(S)).astype(np.int32).reshape(L, u)


def _mask_matrices(L, S, u):
    ck = (L, S, u)
    if ck not in _CONST_CACHE:
        idx = _sample_indices(L, S, u)
        C = np.zeros((L, S), np.float32)
        np.add.at(C, (np.arange(L)[:, None], idx), 1.0)
        A = np.where(C > 0, np.float32(0), np.float32(-np.inf))
        _CONST_CACHE[ck] = (C.T.copy().astype(ml_dtypes.bfloat16),
                            A.T.copy().astype(ml_dtypes.bfloat16))
    return _CONST_CACHE[ck]


def _make_m_topk_body(BH, U):
    def _body(q_ref, k_ref, cf_ref, madd_ref, mt_ref, m_sc):
        L, D = q_ref.shape[1], q_ref.shape[2]
        S = k_ref.shape[1]
        LB = 256
        neg = jnp.float32(-jnp.inf)
        pid = pl.program_id(0)

        @pl.when(pid < BH)
        def _phase_m():
            kk = k_ref[0]
            cdims = (((1,), (1,)), ((), ()))
            for r in range(L // LB):
                sl = slice(LB * r, LB * (r + 1))
                qkT = lax.dot_general(kk, q_ref[0, sl, :], cdims,
                                      preferred_element_type=jnp.float32)
                madd = madd_ref[:, sl].astype(jnp.float32)
                cf = cf_ref[:, sl].astype(jnp.float32)
                mmax = jnp.max(qkT + madd, axis=0, keepdims=True)
                msum = jnp.sum(qkT * cf, axis=0, keepdims=True)
                m_sc[pl.ds(pid, 1), sl] = mmax - msum / jnp.float32(S)

        @pl.when(pid == BH)
        def _phase_topk():
            mv = m_sc[:, :]
            lin = lax.broadcasted_iota(jnp.int32, (BH, L), 1)
            for i in range(U):
                rmax = jnp.max(mv, axis=1, keepdims=True)
                idxc = jnp.min(jnp.where(mv == rmax, lin, S),
                               axis=1, keepdims=True)
                mt_ref[:, pl.ds(i, 1), :] = jnp.broadcast_to(
                    idxc.astype(jnp.float32)[:, :, None], (BH, 1, 128))
                mv = jnp.where(lin == idxc, neg, mv)

    return _body


def _attn_body(q_ref, k_ref, v_ref, mt_ref, o_ref):
    D = q_ref.shape[2]
    S = k_ref.shape[1]
    U = mt_ref.shape[1]
    neg = jnp.float32(-jnp.inf)
    scale = jnp.float32(1.0 / math.sqrt(D))
    kk = k_ref[0]
    vv = v_ref[0]

    mtopf = mt_ref[0, :, 0:1]
    colf = lax.broadcasted_iota(jnp.int32, (U, S), 1).astype(jnp.float32)
    oh = (colf == mtopf).astype(jnp.float32)

    qsel = lax.dot_general(oh, q_ref[0], (((1,), (0,)), ((), ())),
                           preferred_element_type=jnp.float32)
    sc = lax.dot_general(qsel, kk, (((1,), (1,)), ((), ())),
                         preferred_element_type=jnp.float32) * scale
    scm = jnp.where(colf <= mtopf, sc, neg)
    rmax = jnp.max(scm, axis=1, keepdims=True)
    p = jnp.exp(scm - rmax)
    attn = p / jnp.sum(p, axis=1, keepdims=True)
    upd = lax.dot_general(attn, vv, (((1,), (0,)), ((), ())),
                          preferred_element_type=jnp.float32)

    tri = (lax.broadcasted_iota(jnp.int32, (D, D), 0)
           <= lax.broadcasted_iota(jnp.int32, (D, D), 1))
    T = jnp.where(tri, jnp.float32(1.0), jnp.float32(0.0))
    cumvT = lax.dot_general(T, vv, (((0,), (1,)), ((), ())),
                            preferred_element_type=jnp.float32)

    merged = jnp.concatenate([upd, jnp.ones((U, 1), jnp.float32)], axis=1)
    fullT = lax.dot_general(merged, oh, (((0,), (0,)), ((), ())),
                            preferred_element_type=jnp.float32)
    o_ref[0, 0, :, :] = jnp.where(fullT[D:D + 1, :] > 0.5, fullT[:D, :],
                                  cumvT)


def kernel(queries, keys, values, attn_mask):
    B, L, H, D = queries.shape
    S = keys.shape[1]
    BH = B * H
    factor = 5
    U = factor * int(np.ceil(np.log(S)))
    u = factor * int(np.ceil(np.log(L)))
    Cnp, Anp = _mask_matrices(L, S, u)
    CF = jnp.asarray(Cnp)
    MADD = jnp.asarray(Anp)

    q = queries.reshape(BH, L, D)
    k = keys.reshape(BH, S, D)
    v = values.reshape(BH, S, D)

    mt = pl.pallas_call(
        _make_m_topk_body(BH, U),
        grid=(BH + 1,),
        in_specs=[
            pl.BlockSpec((1, L, D), lambda j: (jnp.minimum(j, BH - 1), 0, 0)),
            pl.BlockSpec((1, S, D), lambda j: (jnp.minimum(j, BH - 1), 0, 0)),
            pl.BlockSpec((S, L), lambda j: (0, 0)),
            pl.BlockSpec((S, L), lambda j: (0, 0)),
        ],
        out_specs=pl.BlockSpec((BH, U, 128), lambda j: (0, 0, 0)),
        out_shape=jax.ShapeDtypeStruct((BH, U, 128), jnp.float32),
        scratch_shapes=[pltpu.VMEM((BH, L), jnp.float32)],
    )(q, k, CF, MADD)

    out = pl.pallas_call(
        _attn_body,
        grid=(BH,),
        in_specs=[
            pl.BlockSpec((1, L, D), lambda j: (j, 0, 0)),
            pl.BlockSpec((1, S, D), lambda j: (j, 0, 0)),
            pl.BlockSpec((1, S, D), lambda j: (j, 0, 0)),
            pl.BlockSpec((1, U, 128), lambda j: (j, 0, 0)),
        ],
        out_specs=pl.BlockSpec(
            (1, 1, D, S), lambda j: (j // H, j % H, 0, 0)),
        out_shape=jax.ShapeDtypeStruct((B, H, D, S), jnp.float32),
    )(q, k, v, mt)
    return jnp.swapaxes(out, 2, 3)
